# baseline (speedup 1.0000x reference)
import functools

import jax
import jax.numpy as jnp
from jax import lax
from jax.experimental import pallas as pl
from jax.experimental.pallas import tpu as pltpu
from jax.experimental.pallas import tpu_sc as plsc

B = 16384
DMF = 16
DMLP = 32
NC = 2
NS = 16
NW = NC * NS
BPW = B // NW
NROWS = 1000000
NTB = (NROWS // 128) * 128
NTAIL = NROWS - NTB
RBMAX = NTB // 128 - 1
SLOTS = 6


def _sc_gather_body(uidx_hbm, iidx_hbm, mfuT, mfiT, mluT, mliT,
                    tfu_hbm, tfi_hbm, tmu_hbm, tmi_hbm,
                    out_mluT, out_mliT, out_mfpT,
                    uidx_v, iidx_v, fmu, fmi, ffu, ffi,
                    gmu, gmi, gf, tmu_v, tmi_v, tfu_v, tfi_v, sems):
    wid = lax.axis_index("s") * NC + lax.axis_index("c")
    base = wid * BPW
    pltpu.sync_copy(uidx_hbm.at[wid], uidx_v.at[pl.ds(0, BPW)])
    pltpu.sync_copy(iidx_hbm.at[wid], iidx_v.at[pl.ds(0, BPW)])
    pltpu.sync_copy(tmu_hbm, tmu_v)
    pltpu.sync_copy(tmi_hbm, tmi_v)
    pltpu.sync_copy(tfu_hbm, tfu_v)
    pltpu.sync_copy(tfi_hbm, tfi_v)

    rows = lax.iota(jnp.int32, 16)

    def fire(t):
        ru = uidx_v[pl.ds(t, 16)][0]
        ri = iidx_v[pl.ds(t, 16)][0]
        s = lax.rem(t, SLOTS)
        bu = pl.multiple_of(lax.min(ru >> 7, RBMAX) * 128, 128)
        bi = pl.multiple_of(lax.min(ri >> 7, RBMAX) * 128, 128)
        pltpu.async_copy(mluT.at[:, pl.ds(bu, 128)], fmu.at[s], sems.at[s])
        pltpu.async_copy(mliT.at[:, pl.ds(bi, 128)], fmi.at[s], sems.at[s])
        pltpu.async_copy(mfuT.at[:, pl.ds(bu, 128)], ffu.at[s], sems.at[s])
        pltpu.async_copy(mfiT.at[:, pl.ds(bi, 128)], ffi.at[s], sems.at[s])

    for t0 in range(SLOTS):
        fire(t0)

    def body(t, carry):
        s = lax.rem(t, SLOTS)
        pltpu.make_async_copy(mluT.at[:, pl.ds(0, 128)], fmu.at[s], sems.at[s]).wait()
        pltpu.make_async_copy(mliT.at[:, pl.ds(0, 128)], fmi.at[s], sems.at[s]).wait()
        pltpu.make_async_copy(mfuT.at[:, pl.ds(0, 128)], ffu.at[s], sems.at[s]).wait()
        pltpu.make_async_copy(mfiT.at[:, pl.ds(0, 128)], ffi.at[s], sems.at[s]).wait()
        ru = uidx_v[pl.ds(t, 16)][0]
        ri = iidx_v[pl.ds(t, 16)][0]
        sv = jnp.full((16,), s, jnp.int32)
        colv = jnp.full((16,), t, jnp.int32)
        ruv = jnp.full((16,), ru, jnp.int32)
        riv = jnp.full((16,), ri, jnp.int32)
        lu = ruv & 127
        li = riv & 127
        mu = ruv < NTB
        mi = riv < NTB
        tlu = jnp.maximum(ruv - NTB, 0)
        tli = jnp.maximum(riv - NTB, 0)
        for h in range(DMLP // 16):
            r16 = rows + h * 16
            vm = plsc.load_gather(fmu, [sv, r16, lu])
            vt = plsc.load_gather(tmu_v, [r16, tlu])
            plsc.store_scatter(gmu, [r16, colv], jnp.where(mu, vm, vt))
            wm = plsc.load_gather(fmi, [sv, r16, li])
            wt = plsc.load_gather(tmi_v, [r16, tli])
            plsc.store_scatter(gmi, [r16, colv], jnp.where(mi, wm, wt))
        pu = jnp.where(mu, plsc.load_gather(ffu, [sv, rows, lu]),
                       plsc.load_gather(tfu_v, [rows, tlu]))
        pi = jnp.where(mi, plsc.load_gather(ffi, [sv, rows, li]),
                       plsc.load_gather(tfi_v, [rows, tli]))
        plsc.store_scatter(gf, [rows, colv], pu * pi)

        @pl.when(t + SLOTS < BPW)
        def _():
            fire(t + SLOTS)
        return carry

    lax.fori_loop(0, BPW, body, 0)

    csl = pl.ds(base, BPW)
    pltpu.sync_copy(gmu, out_mluT.at[:, csl])
    pltpu.sync_copy(gmi, out_mliT.at[:, csl])
    pltpu.sync_copy(gf, out_mfpT.at[:, csl])


_sc_gather = functools.partial(
    pl.kernel,
    mesh=plsc.VectorSubcoreMesh(core_axis_name="c", subcore_axis_name="s"),
    out_type=[
        jax.ShapeDtypeStruct((DMLP, B), jnp.float32),
        jax.ShapeDtypeStruct((DMLP, B), jnp.float32),
        jax.ShapeDtypeStruct((DMF, B), jnp.float32),
    ],
    scratch_types=[
        pltpu.VMEM((BPW + 32,), jnp.int32),
        pltpu.VMEM((BPW + 32,), jnp.int32),
        pltpu.VMEM((SLOTS, DMLP, 128), jnp.float32),
        pltpu.VMEM((SLOTS, DMLP, 128), jnp.float32),
        pltpu.VMEM((SLOTS, DMF, 128), jnp.float32),
        pltpu.VMEM((SLOTS, DMF, 128), jnp.float32),
        pltpu.VMEM((DMLP, BPW), jnp.float32),
        pltpu.VMEM((DMLP, BPW), jnp.float32),
        pltpu.VMEM((DMF, BPW), jnp.float32),
        pltpu.VMEM((DMLP, NTAIL), jnp.float32),
        pltpu.VMEM((DMLP, NTAIL), jnp.float32),
        pltpu.VMEM((DMF, NTAIL), jnp.float32),
        pltpu.VMEM((DMF, NTAIL), jnp.float32),
        pltpu.SemaphoreType.DMA((SLOTS,)),
    ],
    compiler_params=pltpu.CompilerParams(needs_layout_passes=False,
                                         use_tc_tiling_on_sc=True),
)(_sc_gather_body)


def _mlp_body(xuT_ref, xiT_ref, mfpT_ref, w0uT_ref, w0iT_ref, b0_ref,
              w1T_ref, b1_ref, w2T_ref, b2_ref, wnmT_ref, wnhT_ref, bn_ref,
              out_ref):
    h = jnp.dot(w0uT_ref[...], xuT_ref[...], preferred_element_type=jnp.float32)
    h += jnp.dot(w0iT_ref[...], xiT_ref[...], preferred_element_type=jnp.float32)
    h = jnp.maximum(h + b0_ref[...], 0.0)
    h = jnp.maximum(jnp.dot(w1T_ref[...], h, preferred_element_type=jnp.float32)
                    + b1_ref[...], 0.0)
    h = jnp.maximum(jnp.dot(w2T_ref[...], h, preferred_element_type=jnp.float32)
                    + b2_ref[...], 0.0)
    logit = jnp.dot(wnmT_ref[...], mfpT_ref[...],
                    preferred_element_type=jnp.float32)
    logit += jnp.dot(wnhT_ref[...], h, preferred_element_type=jnp.float32)
    logit += bn_ref[...]
    out_ref[...] = 1.0 / (1.0 + jnp.exp(-logit))


def _mlp_call(xuT, xiT, mfpT, w0uT, w0iT, b0, w1T, b1, w2T, b2,
              wnmT, wnhT, bn):
    BT = 2048
    grid = (B // BT,)
    col_spec = lambda d: pl.BlockSpec((d, BT), lambda i: (0, i))
    full = lambda a, b: pl.BlockSpec((a, b), lambda i: (0, 0))
    return pl.pallas_call(
        _mlp_body,
        grid=grid,
        in_specs=[
            col_spec(DMLP), col_spec(DMLP), col_spec(DMF),
            full(32, DMLP), full(32, DMLP), full(32, 1),
            full(16, 32), full(16, 1),
            full(8, 16), full(8, 1),
            full(1, DMF), full(1, 8), full(1, 1),
        ],
        out_specs=pl.BlockSpec((1, BT), lambda i: (0, i)),
        out_shape=jax.ShapeDtypeStruct((1, B), jnp.float32),
    )(xuT, xiT, mfpT, w0uT, w0iT, b0, w1T, b1, w2T, b2, wnmT, wnhT, bn)


@jax.jit
def kernel(user_indices, item_indices, mf_user_table, mf_item_table,
           mlp_user_table, mlp_item_table, W0, b0, W1, b1, W2, b2, Wn, bn):
    pu = jnp.argsort(user_indices).astype(jnp.int32)
    pi = jnp.argsort(item_indices).astype(jnp.int32)
    ipu = jnp.argsort(pu).astype(jnp.int32)
    ipi = jnp.argsort(pi).astype(jnp.int32)
    su = user_indices[pu].astype(jnp.int32)
    si = item_indices[pi].astype(jnp.int32)
    probe = jnp.minimum(su[0] + si[0] + ipu[0] + ipi[0], 0).astype(jnp.int32)
    uidx = (user_indices.astype(jnp.int32) + probe).reshape(NW, BPW)
    iidx = item_indices.astype(jnp.int32).reshape(NW, BPW)
    mluT, mliT, mfpT = _sc_gather(
        uidx, iidx,
        mf_user_table.T, mf_item_table.T,
        mlp_user_table.T, mlp_item_table.T,
        mf_user_table[NTB:].T, mf_item_table[NTB:].T,
        mlp_user_table[NTB:].T, mlp_item_table[NTB:].T,
    )
    outT = _mlp_call(mluT, mliT, mfpT,
                     W0[:DMLP].T, W0[DMLP:].T, b0.reshape(32, 1),
                     W1.T, b1.reshape(16, 1), W2.T, b2.reshape(8, 1),
                     Wn[:DMF].T, Wn[DMF:].T, bn.reshape(1, 1))
    return outT.reshape(B, 1)


# speedup vs baseline: 1.3967x; 1.3967x over previous
import functools

import jax
import jax.numpy as jnp
from jax import lax
from jax.experimental import pallas as pl
from jax.experimental.pallas import tpu as pltpu
from jax.experimental.pallas import tpu_sc as plsc

B = 16384
DMF = 16
DMLP = 32
NC = 2
NS = 16
NW = NC * NS
BPW = B // NW
NROWS = 1000000
NTB = (NROWS // 128) * 128
NTAIL = NROWS - NTB
RBMAX = NTB // 128 - 1
SLOTS = 6
PCH = BPW // 128


def _sc_gather_body(su_hbm, si_hbm, pu_hbm, pi_hbm,
                    mfuT, mfiT, mluT, mliT,
                    tfu_hbm, tfi_hbm, tmu_hbm, tmi_hbm,
                    out_u, out_i,
                    sidx_v, perm_v, fml, fmf, rowbuf, tml_v, tmf_v, sems,
                    osem):
    wid = lax.axis_index("s") * NC + lax.axis_index("c")
    rows = lax.iota(jnp.int32, 16)

    def run_phase(idx_hbm, perm_hbm, mlT, mfT, tml_hbm, tmf_hbm, out_hbm):
        pltpu.sync_copy(idx_hbm.at[wid], sidx_v.at[pl.ds(0, BPW)])
        pltpu.sync_copy(perm_hbm.at[pl.ds(wid * PCH, PCH)], perm_v)
        pltpu.sync_copy(tml_hbm, tml_v)
        pltpu.sync_copy(tmf_hbm, tmf_v)

        def blk_at(p):
            return lax.min(sidx_v[pl.ds(p, 16)][0] >> 7, RBMAX)

        def fire(t, slot):
            bu = pl.multiple_of(blk_at(t) * 128, 128)
            pltpu.async_copy(mlT.at[:, pl.ds(bu, 128)], fml.at[slot],
                             sems.at[slot])
            pltpu.async_copy(mfT.at[:, pl.ds(bu, 128)], fmf.at[slot],
                             sems.at[slot])

        ca = jnp.int32(0)
        for t0 in range(SLOTS):
            if t0 == 0:
                f = jnp.bool_(True)
            else:
                f = blk_at(t0) != blk_at(t0 - 1)
            slot = lax.rem(ca, SLOTS)

            @pl.when(f)
            def _():
                fire(t0, slot)
            ca = ca + f.astype(jnp.int32)

        def body(t, carry):
            c, ca = carry
            f = jnp.logical_or(t == 0,
                               blk_at(t) != blk_at(lax.max(t - 1, 0)))
            c = c + f.astype(jnp.int32)
            sl = lax.rem(c - 1, SLOTS)

            @pl.when(f)
            def _():
                pltpu.make_async_copy(mlT.at[:, pl.ds(0, 128)], fml.at[sl],
                                      sems.at[sl]).wait()
                pltpu.make_async_copy(mfT.at[:, pl.ds(0, 128)], fmf.at[sl],
                                      sems.at[sl]).wait()

            r = sidx_v[pl.ds(t, 16)][0]
            rv = jnp.full((16,), r, jnp.int32)
            slv = jnp.full((16,), sl, jnp.int32)
            lane = rv & 127
            main = rv < NTB
            tl = jnp.maximum(rv - NTB, 0)
            for h in range(DMLP // 16):
                r16 = rows + h * 16
                vm = plsc.load_gather(fml, [slv, r16, lane])
                vt = plsc.load_gather(tml_v, [r16, tl])
                rowbuf[t, pl.ds(h * 16, 16)] = jnp.where(main, vm, vt)
            vf = jnp.where(main, plsc.load_gather(fmf, [slv, rows, lane]),
                           plsc.load_gather(tmf_v, [rows, tl]))
            rowbuf[t, pl.ds(DMLP, DMF)] = vf

            tf = t + SLOTS
            fa = jnp.logical_and(tf < BPW, blk_at(tf) != blk_at(tf - 1))

            @pl.when(fa)
            def _():
                fire(tf, lax.rem(ca, SLOTS))
            return (c, ca + fa.astype(jnp.int32))

        lax.fori_loop(0, BPW, body, (jnp.int32(0), ca))

        scat = []
        for j in range(PCH):
            scat.append(pltpu.async_copy(
                rowbuf.at[pl.ds(j * 128, 128)], out_hbm.at[perm_v.at[j]],
                osem))
        for s in scat:
            s.wait()

    run_phase(su_hbm, pu_hbm, mluT, mfuT, tmu_hbm, tfu_hbm, out_u)
    run_phase(si_hbm, pi_hbm, mliT, mfiT, tmi_hbm, tfi_hbm, out_i)


_sc_gather = functools.partial(
    pl.kernel,
    mesh=plsc.VectorSubcoreMesh(core_axis_name="c", subcore_axis_name="s"),
    out_type=[
        jax.ShapeDtypeStruct((B, 128), jnp.float32),
        jax.ShapeDtypeStruct((B, 128), jnp.float32),
    ],
    scratch_types=[
        pltpu.VMEM((BPW + 32,), jnp.int32),
        pltpu.VMEM((PCH, 128), jnp.int32),
        pltpu.VMEM((SLOTS, DMLP, 128), jnp.float32),
        pltpu.VMEM((SLOTS, DMF, 128), jnp.float32),
        pltpu.VMEM((BPW, 128), jnp.float32),
        pltpu.VMEM((DMLP, NTAIL), jnp.float32),
        pltpu.VMEM((DMF, NTAIL), jnp.float32),
        pltpu.SemaphoreType.DMA((SLOTS,)),
        pltpu.SemaphoreType.DMA,
    ],
    compiler_params=pltpu.CompilerParams(needs_layout_passes=False,
                                         use_tc_tiling_on_sc=True),
)(_sc_gather_body)


def _mlp_body(u_ref, i_ref, w0u_ref, w0i_ref, b0_ref, w1_ref, b1_ref,
              w2_ref, b2_ref, wnm_ref, wnh_ref, bn_ref, out_ref):
    bu = u_ref[...]
    bi = i_ref[...]
    xu = bu[:, 0:DMLP]
    xi = bi[:, 0:DMLP]
    mfp = bu[:, DMLP:DMLP + DMF] * bi[:, DMLP:DMLP + DMF]
    h = jnp.dot(xu, w0u_ref[...], preferred_element_type=jnp.float32)
    h += jnp.dot(xi, w0i_ref[...], preferred_element_type=jnp.float32)
    h = jnp.maximum(h + b0_ref[...], 0.0)
    h = jnp.maximum(jnp.dot(h, w1_ref[...], preferred_element_type=jnp.float32)
                    + b1_ref[...], 0.0)
    h = jnp.maximum(jnp.dot(h, w2_ref[...], preferred_element_type=jnp.float32)
                    + b2_ref[...], 0.0)
    logit = jnp.dot(mfp, wnm_ref[...], preferred_element_type=jnp.float32)
    logit += jnp.dot(h, wnh_ref[...], preferred_element_type=jnp.float32)
    logit += bn_ref[...]
    out_ref[...] = 1.0 / (1.0 + jnp.exp(-logit))


def _mlp_call(gu, gi, w0u, w0i, b0, w1, b1, w2, b2, wnm, wnh, bn):
    BT = 2048
    grid = (B // BT,)
    row_spec = pl.BlockSpec((BT, 128), lambda i: (i, 0))
    full = lambda a, b: pl.BlockSpec((a, b), lambda i: (0, 0))
    return pl.pallas_call(
        _mlp_body,
        grid=grid,
        in_specs=[
            row_spec, row_spec,
            full(DMLP, 32), full(DMLP, 32), full(1, 32),
            full(32, 16), full(1, 16),
            full(16, 8), full(1, 8),
            full(DMF, 1), full(8, 1), full(1, 1),
        ],
        out_specs=pl.BlockSpec((BT, 1), lambda i: (i, 0)),
        out_shape=jax.ShapeDtypeStruct((B, 1), jnp.float32),
    )(gu, gi, w0u, w0i, b0, w1, b1, w2, b2, wnm, wnh, bn)


@jax.jit
def kernel(user_indices, item_indices, mf_user_table, mf_item_table,
           mlp_user_table, mlp_item_table, W0, b0, W1, b1, W2, b2, Wn, bn):
    pu = jnp.argsort(user_indices).astype(jnp.int32)
    pi = jnp.argsort(item_indices).astype(jnp.int32)
    su = user_indices[pu].astype(jnp.int32).reshape(NW, BPW)
    si = item_indices[pi].astype(jnp.int32).reshape(NW, BPW)
    gu, gi = _sc_gather(
        su, si,
        pu.reshape(B // 128, 128), pi.reshape(B // 128, 128),
        mf_user_table.T, mf_item_table.T,
        mlp_user_table.T, mlp_item_table.T,
        mf_user_table[NTB:].T, mf_item_table[NTB:].T,
        mlp_user_table[NTB:].T, mlp_item_table[NTB:].T,
    )
    return _mlp_call(gu, gi,
                     W0[:DMLP], W0[DMLP:], b0.reshape(1, 32),
                     W1, b1.reshape(1, 16), W2, b2.reshape(1, 8),
                     Wn[:DMF], Wn[DMF:], bn.reshape(1, 1))


# speedup vs baseline: 1.4572x; 1.0433x over previous
import functools

import jax
import jax.numpy as jnp
from jax import lax
from jax.experimental import pallas as pl
from jax.experimental.pallas import tpu as pltpu
from jax.experimental.pallas import tpu_sc as plsc

B = 16384
DMF = 16
DMLP = 32
NC = 2
NS = 16
NW = NC * NS
BPW = B // NW
NROWS = 1000000
NTB = (NROWS // 128) * 128
NTAIL = NROWS - NTB
RBMAX = NTB // 128 - 1
SLOTS = 6
PCH = BPW // 128


def _sc_gather_body(su_hbm, si_hbm, pu_hbm, pi_hbm,
                    mfuT, mfiT, mluT, mliT,
                    tfu_hbm, tfi_hbm, tmu_hbm, tmi_hbm,
                    out_u, out_i,
                    sidx_v, perm_v, fml, fmf, rowbuf, tml_v, tmf_v, sems,
                    osem):
    wid = lax.axis_index("s") * NC + lax.axis_index("c")
    rows = lax.iota(jnp.int32, 16)

    def run_phase(idx_hbm, perm_hbm, mlT, mfT, tml_hbm, tmf_hbm, out_hbm):
        pltpu.sync_copy(idx_hbm.at[wid], sidx_v.at[pl.ds(0, BPW)])
        pltpu.sync_copy(perm_hbm.at[pl.ds(wid * PCH, PCH)], perm_v)
        pltpu.sync_copy(tml_hbm, tml_v)
        pltpu.sync_copy(tmf_hbm, tmf_v)

        def blk_at(p):
            return lax.min(sidx_v[pl.ds(p, 16)][0] >> 7, RBMAX)

        def fire(t, slot):
            bu = pl.multiple_of(blk_at(t) * 128, 128)
            pltpu.async_copy(mlT.at[:, pl.ds(bu, 128)], fml.at[slot],
                             sems.at[slot])
            pltpu.async_copy(mfT.at[:, pl.ds(bu, 128)], fmf.at[slot],
                             sems.at[slot])

        ca = jnp.int32(0)
        for t0 in range(SLOTS):
            if t0 == 0:
                f = jnp.bool_(True)
            else:
                f = blk_at(t0) != blk_at(t0 - 1)
            slot = lax.rem(ca, SLOTS)

            @pl.when(f)
            def _():
                fire(t0, slot)
            ca = ca + f.astype(jnp.int32)

        def body(t, carry):
            c, ca = carry
            f = jnp.logical_or(t == 0,
                               blk_at(t) != blk_at(lax.max(t - 1, 0)))
            c = c + f.astype(jnp.int32)
            sl = lax.rem(c - 1, SLOTS)

            @pl.when(f)
            def _():
                pltpu.make_async_copy(mlT.at[:, pl.ds(0, 128)], fml.at[sl],
                                      sems.at[sl]).wait()
                pltpu.make_async_copy(mfT.at[:, pl.ds(0, 128)], fmf.at[sl],
                                      sems.at[sl]).wait()

            r = sidx_v[pl.ds(t, 16)][0]
            rv = jnp.full((16,), r, jnp.int32)
            slv = jnp.full((16,), sl, jnp.int32)
            lane = rv & 127
            main = rv < NTB
            tl = jnp.maximum(rv - NTB, 0)
            for h in range(DMLP // 16):
                r16 = rows + h * 16
                vm = plsc.load_gather(fml, [slv, r16, lane])
                vt = plsc.load_gather(tml_v, [r16, tl])
                rowbuf[t, pl.ds(h * 16, 16)] = jnp.where(main, vm, vt)
            vf = jnp.where(main, plsc.load_gather(fmf, [slv, rows, lane]),
                           plsc.load_gather(tmf_v, [rows, tl]))
            rowbuf[t, pl.ds(DMLP, DMF)] = vf

            tf = t + SLOTS
            fa = jnp.logical_and(tf < BPW, blk_at(tf) != blk_at(tf - 1))

            @pl.when(fa)
            def _():
                fire(tf, lax.rem(ca, SLOTS))
            return (c, ca + fa.astype(jnp.int32))

        def body2(k, carry):
            carry = body(2 * k, carry)
            return body(2 * k + 1, carry)

        lax.fori_loop(0, BPW // 2, body2, (jnp.int32(0), ca))

        scat = []
        for j in range(PCH):
            scat.append(pltpu.async_copy(
                rowbuf.at[pl.ds(j * 128, 128)], out_hbm.at[perm_v.at[j]],
                osem))
        for s in scat:
            s.wait()

    run_phase(su_hbm, pu_hbm, mluT, mfuT, tmu_hbm, tfu_hbm, out_u)
    run_phase(si_hbm, pi_hbm, mliT, mfiT, tmi_hbm, tfi_hbm, out_i)


_sc_gather = functools.partial(
    pl.kernel,
    mesh=plsc.VectorSubcoreMesh(core_axis_name="c", subcore_axis_name="s"),
    out_type=[
        jax.ShapeDtypeStruct((B, 128), jnp.float32),
        jax.ShapeDtypeStruct((B, 128), jnp.float32),
    ],
    scratch_types=[
        pltpu.VMEM((BPW + 32,), jnp.int32),
        pltpu.VMEM((PCH, 128), jnp.int32),
        pltpu.VMEM((SLOTS, DMLP, 128), jnp.float32),
        pltpu.VMEM((SLOTS, DMF, 128), jnp.float32),
        pltpu.VMEM((BPW, 128), jnp.float32),
        pltpu.VMEM((DMLP, NTAIL), jnp.float32),
        pltpu.VMEM((DMF, NTAIL), jnp.float32),
        pltpu.SemaphoreType.DMA((SLOTS,)),
        pltpu.SemaphoreType.DMA,
    ],
    compiler_params=pltpu.CompilerParams(needs_layout_passes=False,
                                         use_tc_tiling_on_sc=True),
)(_sc_gather_body)


def _mlp_body(u_ref, i_ref, w0u_ref, w0i_ref, b0_ref, w1_ref, b1_ref,
              w2_ref, b2_ref, wnm_ref, wnh_ref, bn_ref, out_ref):
    bu = u_ref[...]
    bi = i_ref[...]
    xu = bu[:, 0:DMLP]
    xi = bi[:, 0:DMLP]
    mfp = bu[:, DMLP:DMLP + DMF] * bi[:, DMLP:DMLP + DMF]
    h = jnp.dot(xu, w0u_ref[...], preferred_element_type=jnp.float32)
    h += jnp.dot(xi, w0i_ref[...], preferred_element_type=jnp.float32)
    h = jnp.maximum(h + b0_ref[...], 0.0)
    h = jnp.maximum(jnp.dot(h, w1_ref[...], preferred_element_type=jnp.float32)
                    + b1_ref[...], 0.0)
    h = jnp.maximum(jnp.dot(h, w2_ref[...], preferred_element_type=jnp.float32)
                    + b2_ref[...], 0.0)
    logit = jnp.dot(mfp, wnm_ref[...], preferred_element_type=jnp.float32)
    logit += jnp.dot(h, wnh_ref[...], preferred_element_type=jnp.float32)
    logit += bn_ref[...]
    out_ref[...] = 1.0 / (1.0 + jnp.exp(-logit))


def _mlp_call(gu, gi, w0u, w0i, b0, w1, b1, w2, b2, wnm, wnh, bn):
    BT = 2048
    grid = (B // BT,)
    row_spec = pl.BlockSpec((BT, 128), lambda i: (i, 0))
    full = lambda a, b: pl.BlockSpec((a, b), lambda i: (0, 0))
    return pl.pallas_call(
        _mlp_body,
        grid=grid,
        in_specs=[
            row_spec, row_spec,
            full(DMLP, 32), full(DMLP, 32), full(1, 32),
            full(32, 16), full(1, 16),
            full(16, 8), full(1, 8),
            full(DMF, 1), full(8, 1), full(1, 1),
        ],
        out_specs=pl.BlockSpec((BT, 1), lambda i: (i, 0)),
        out_shape=jax.ShapeDtypeStruct((B, 1), jnp.float32),
    )(gu, gi, w0u, w0i, b0, w1, b1, w2, b2, wnm, wnh, bn)


@jax.jit
def kernel(user_indices, item_indices, mf_user_table, mf_item_table,
           mlp_user_table, mlp_item_table, W0, b0, W1, b1, W2, b2, Wn, bn):
    ar = lax.iota(jnp.int32, B)
    su, pu = lax.sort((user_indices.astype(jnp.int32), ar), num_keys=1)
    si, pi = lax.sort((item_indices.astype(jnp.int32), ar), num_keys=1)
    su = su.reshape(NW, BPW)
    si = si.reshape(NW, BPW)
    gu, gi = _sc_gather(
        su, si,
        pu.reshape(B // 128, 128), pi.reshape(B // 128, 128),
        mf_user_table.T, mf_item_table.T,
        mlp_user_table.T, mlp_item_table.T,
        mf_user_table[NTB:].T, mf_item_table[NTB:].T,
        mlp_user_table[NTB:].T, mlp_item_table[NTB:].T,
    )
    return _mlp_call(gu, gi,
                     W0[:DMLP], W0[DMLP:], b0.reshape(1, 32),
                     W1, b1.reshape(1, 16), W2, b2.reshape(1, 8),
                     Wn[:DMF], Wn[DMF:], bn.reshape(1, 1))


# speedup vs baseline: 1.6386x; 1.1245x over previous
import functools

import jax
import jax.numpy as jnp
from jax import lax
from jax.experimental import pallas as pl
from jax.experimental.pallas import tpu as pltpu
from jax.experimental.pallas import tpu_sc as plsc

B = 16384
DMF = 16
DMLP = 32
NC = 2
NS = 16
NW = NC * NS
BPW = B // NW
NROWS = 1000000
NTB = (NROWS // 128) * 128
NTAIL = NROWS - NTB
RBMAX = NTB // 128 - 1
SLOTS = 8
PCH = BPW // 128


def _sc_gather_body(su_hbm, si_hbm, pu_hbm, pi_hbm,
                    mfuT, mfiT, mluT, mliT,
                    tfu_hbm, tfi_hbm, tmu_hbm, tmi_hbm,
                    out_u, out_i,
                    sidx_v, perm_v, fml, fmf, rowbuf, tml_v, tmf_v, sems,
                    osem):
    wid = lax.axis_index("s") * NC + lax.axis_index("c")
    rows = lax.iota(jnp.int32, 16)

    def run_phase(idx_hbm, perm_hbm, mlT, mfT, tml_hbm, tmf_hbm, out_hbm):
        pltpu.sync_copy(idx_hbm.at[wid], sidx_v.at[pl.ds(0, BPW)])
        pltpu.sync_copy(perm_hbm.at[pl.ds(wid * PCH, PCH)], perm_v)
        pltpu.sync_copy(tml_hbm, tml_v)
        pltpu.sync_copy(tmf_hbm, tmf_v)

        def blk_at(p):
            return lax.min(sidx_v[pl.ds(p, 16)][0] >> 7, RBMAX)

        def fire(t, slot):
            bu = pl.multiple_of(blk_at(t) * 128, 128)
            pltpu.async_copy(mlT.at[:, pl.ds(bu, 128)], fml.at[slot],
                             sems.at[slot])
            pltpu.async_copy(mfT.at[:, pl.ds(bu, 128)], fmf.at[slot],
                             sems.at[slot])

        ca = jnp.int32(0)
        for t0 in range(SLOTS):
            if t0 == 0:
                f = jnp.bool_(True)
            else:
                f = blk_at(t0) != blk_at(t0 - 1)
            slot = lax.rem(ca, SLOTS)

            @pl.when(f)
            def _():
                fire(t0, slot)
            ca = ca + f.astype(jnp.int32)

        def body(t, carry):
            c, ca, bprev, bpa = carry
            bt = blk_at(t)
            f = jnp.logical_or(t == 0, bt != bprev)
            c = c + f.astype(jnp.int32)
            sl = lax.rem(c - 1, SLOTS)

            @pl.when(f)
            def _():
                pltpu.make_async_copy(mlT.at[:, pl.ds(0, 128)], fml.at[sl],
                                      sems.at[sl]).wait()
                pltpu.make_async_copy(mfT.at[:, pl.ds(0, 128)], fmf.at[sl],
                                      sems.at[sl]).wait()

            r = sidx_v[pl.ds(t, 16)][0]
            rv = jnp.full((16,), r, jnp.int32)
            slv = jnp.full((16,), sl, jnp.int32)
            lane = rv & 127
            main = rv < NTB
            tl = jnp.maximum(rv - NTB, 0)
            for h in range(DMLP // 16):
                r16 = rows + h * 16
                vm = plsc.load_gather(fml, [slv, r16, lane])
                vt = plsc.load_gather(tml_v, [r16, tl])
                rowbuf[t, pl.ds(h * 16, 16)] = jnp.where(main, vm, vt)
            vf = jnp.where(main, plsc.load_gather(fmf, [slv, rows, lane]),
                           plsc.load_gather(tmf_v, [rows, tl]))
            rowbuf[t, pl.ds(DMLP, DMF)] = vf

            tf = t + SLOTS
            bfa = blk_at(tf)
            fa = jnp.logical_and(tf < BPW, bfa != bpa)

            @pl.when(fa)
            def _():
                fire(tf, lax.rem(ca, SLOTS))
            return (c, ca + fa.astype(jnp.int32), bt, bfa)

        def body2(k, carry):
            carry = body(2 * k, carry)
            return body(2 * k + 1, carry)

        lax.fori_loop(0, BPW // 2, body2,
                      (jnp.int32(0), ca, jnp.int32(-1), blk_at(SLOTS - 1)))

        scat = []
        for j in range(PCH):
            scat.append(pltpu.async_copy(
                rowbuf.at[pl.ds(j * 128, 128)], out_hbm.at[perm_v.at[j]],
                osem))
        for s in scat:
            s.wait()

    run_phase(su_hbm, pu_hbm, mluT, mfuT, tmu_hbm, tfu_hbm, out_u)
    run_phase(si_hbm, pi_hbm, mliT, mfiT, tmi_hbm, tfi_hbm, out_i)


_sc_gather = functools.partial(
    pl.kernel,
    mesh=plsc.VectorSubcoreMesh(core_axis_name="c", subcore_axis_name="s"),
    out_type=[
        jax.ShapeDtypeStruct((B, 128), jnp.float32),
        jax.ShapeDtypeStruct((B, 128), jnp.float32),
    ],
    scratch_types=[
        pltpu.VMEM((BPW + 32,), jnp.int32),
        pltpu.VMEM((PCH, 128), jnp.int32),
        pltpu.VMEM((SLOTS, DMLP, 128), jnp.float32),
        pltpu.VMEM((SLOTS, DMF, 128), jnp.float32),
        pltpu.VMEM((BPW, 128), jnp.float32),
        pltpu.VMEM((DMLP, NTAIL), jnp.float32),
        pltpu.VMEM((DMF, NTAIL), jnp.float32),
        pltpu.SemaphoreType.DMA((SLOTS,)),
        pltpu.SemaphoreType.DMA,
    ],
    compiler_params=pltpu.CompilerParams(needs_layout_passes=False,
                                         use_tc_tiling_on_sc=True),
)(_sc_gather_body)


def _mlp_body(u_ref, i_ref, w0u_ref, w0i_ref, b0_ref, w1_ref, b1_ref,
              w2_ref, b2_ref, wnm_ref, wnh_ref, bn_ref, out_ref):
    bu = u_ref[...]
    bi = i_ref[...]
    xu = bu[:, 0:DMLP]
    xi = bi[:, 0:DMLP]
    mfp = bu[:, DMLP:DMLP + DMF] * bi[:, DMLP:DMLP + DMF]
    h = jnp.dot(xu, w0u_ref[...], preferred_element_type=jnp.float32)
    h += jnp.dot(xi, w0i_ref[...], preferred_element_type=jnp.float32)
    h = jnp.maximum(h + b0_ref[...], 0.0)
    h = jnp.maximum(jnp.dot(h, w1_ref[...], preferred_element_type=jnp.float32)
                    + b1_ref[...], 0.0)
    h = jnp.maximum(jnp.dot(h, w2_ref[...], preferred_element_type=jnp.float32)
                    + b2_ref[...], 0.0)
    logit = jnp.dot(mfp, wnm_ref[...], preferred_element_type=jnp.float32)
    logit += jnp.dot(h, wnh_ref[...], preferred_element_type=jnp.float32)
    logit += bn_ref[...]
    out_ref[...] = 1.0 / (1.0 + jnp.exp(-logit))


def _mlp_call(gu, gi, w0u, w0i, b0, w1, b1, w2, b2, wnm, wnh, bn):
    BT = 2048
    grid = (B // BT,)
    row_spec = pl.BlockSpec((BT, 128), lambda i: (i, 0))
    full = lambda a, b: pl.BlockSpec((a, b), lambda i: (0, 0))
    return pl.pallas_call(
        _mlp_body,
        grid=grid,
        in_specs=[
            row_spec, row_spec,
            full(DMLP, 32), full(DMLP, 32), full(1, 32),
            full(32, 16), full(1, 16),
            full(16, 8), full(1, 8),
            full(DMF, 1), full(8, 1), full(1, 1),
        ],
        out_specs=pl.BlockSpec((BT, 1), lambda i: (i, 0)),
        out_shape=jax.ShapeDtypeStruct((B, 1), jnp.float32),
    )(gu, gi, w0u, w0i, b0, w1, b1, w2, b2, wnm, wnh, bn)


@jax.jit
def kernel(user_indices, item_indices, mf_user_table, mf_item_table,
           mlp_user_table, mlp_item_table, W0, b0, W1, b1, W2, b2, Wn, bn):
    ar = lax.iota(jnp.int32, B)
    su, pu = lax.sort((user_indices.astype(jnp.int32), ar), num_keys=1)
    si, pi = lax.sort((item_indices.astype(jnp.int32), ar), num_keys=1)
    su = su.reshape(NW, BPW)
    si = si.reshape(NW, BPW)
    gu, gi = _sc_gather(
        su, si,
        pu.reshape(B // 128, 128), pi.reshape(B // 128, 128),
        mf_user_table.T, mf_item_table.T,
        mlp_user_table.T, mlp_item_table.T,
        mf_user_table[NTB:].T, mf_item_table[NTB:].T,
        mlp_user_table[NTB:].T, mlp_item_table[NTB:].T,
    )
    return _mlp_call(gu, gi,
                     W0[:DMLP], W0[DMLP:], b0.reshape(1, 32),
                     W1, b1.reshape(1, 16), W2, b2.reshape(1, 8),
                     Wn[:DMF], Wn[DMF:], bn.reshape(1, 1))


# speedup vs baseline: 1.6446x; 1.0037x over previous
import functools

import jax
import jax.numpy as jnp
from jax import lax
from jax.experimental import pallas as pl
from jax.experimental.pallas import tpu as pltpu
from jax.experimental.pallas import tpu_sc as plsc

B = 16384
DMF = 16
DMLP = 32
NC = 2
NS = 16
NW = NC * NS
BPW = B // NW
NROWS = 1000000
NTB = (NROWS // 128) * 128
NTAIL = NROWS - NTB
RBMAX = NTB // 128 - 1
SLOTS = 8
PCH = BPW // 128


def _sc_gather_body(su_hbm, si_hbm, pu_hbm, pi_hbm,
                    mfuT, mfiT, mluT, mliT,
                    tfu_hbm, tfi_hbm, tmu_hbm, tmi_hbm,
                    out_u, out_i,
                    sidx_v, perm_v, fml, fmf, rowbuf, tml_v, tmf_v, sems,
                    osem):
    wid = lax.axis_index("s") * NC + lax.axis_index("c")
    rows = lax.iota(jnp.int32, 16)

    def run_phase(idx_hbm, perm_hbm, mlT, mfT, tml_hbm, tmf_hbm, out_hbm):
        pltpu.sync_copy(idx_hbm.at[wid], sidx_v.at[pl.ds(0, BPW)])
        pltpu.sync_copy(perm_hbm.at[pl.ds(wid * PCH, PCH)], perm_v)
        pltpu.sync_copy(tml_hbm, tml_v)
        pltpu.sync_copy(tmf_hbm, tmf_v)

        def blk_at(p):
            return lax.min(sidx_v[pl.ds(p, 16)][0] >> 7, RBMAX)

        def fire(t, slot):
            bu = pl.multiple_of(blk_at(t) * 128, 128)
            pltpu.async_copy(mlT.at[:, pl.ds(bu, 128)], fml.at[slot],
                             sems.at[slot])
            pltpu.async_copy(mfT.at[:, pl.ds(bu, 128)], fmf.at[slot],
                             sems.at[slot])

        ca = jnp.int32(0)
        for t0 in range(SLOTS):
            if t0 == 0:
                f = jnp.bool_(True)
            else:
                f = blk_at(t0) != blk_at(t0 - 1)
            slot = lax.rem(ca, SLOTS)

            @pl.when(f)
            def _():
                fire(t0, slot)
            ca = ca + f.astype(jnp.int32)

        def body(t, carry):
            c, ca, bprev, bpa = carry
            r = sidx_v[pl.ds(t, 16)][0]
            bt = lax.min(r >> 7, RBMAX)
            f = jnp.logical_or(t == 0, bt != bprev)
            c = c + f.astype(jnp.int32)
            sl = lax.rem(c - 1, SLOTS)

            @pl.when(f)
            def _():
                pltpu.make_async_copy(mlT.at[:, pl.ds(0, 128)], fml.at[sl],
                                      sems.at[sl]).wait()
                pltpu.make_async_copy(mfT.at[:, pl.ds(0, 128)], fmf.at[sl],
                                      sems.at[sl]).wait()

            rv = jnp.full((16,), r, jnp.int32)
            slv = jnp.full((16,), sl, jnp.int32)
            lane = rv & 127
            for h in range(DMLP // 16):
                r16 = rows + h * 16
                rowbuf[t, pl.ds(h * 16, 16)] = plsc.load_gather(
                    fml, [slv, r16, lane])
            rowbuf[t, pl.ds(DMLP, DMF)] = plsc.load_gather(
                fmf, [slv, rows, lane])

            @pl.when(r >= NTB)
            def _():
                tl = rv - NTB
                for h in range(DMLP // 16):
                    r16 = rows + h * 16
                    rowbuf[t, pl.ds(h * 16, 16)] = plsc.load_gather(
                        tml_v, [r16, tl])
                rowbuf[t, pl.ds(DMLP, DMF)] = plsc.load_gather(
                    tmf_v, [rows, tl])

            tf = t + SLOTS
            bfa = blk_at(tf)
            fa = jnp.logical_and(tf < BPW, bfa != bpa)

            @pl.when(fa)
            def _():
                fire(tf, lax.rem(ca, SLOTS))
            return (c, ca + fa.astype(jnp.int32), bt, bfa)

        def body4(k, carry):
            for q in range(4):
                carry = body(4 * k + q, carry)
            return carry

        lax.fori_loop(0, BPW // 4, body4,
                      (jnp.int32(0), ca, jnp.int32(-1), blk_at(SLOTS - 1)))

        scat = []
        for j in range(PCH):
            scat.append(pltpu.async_copy(
                rowbuf.at[pl.ds(j * 128, 128)], out_hbm.at[perm_v.at[j]],
                osem))
        for s in scat:
            s.wait()

    run_phase(su_hbm, pu_hbm, mluT, mfuT, tmu_hbm, tfu_hbm, out_u)
    run_phase(si_hbm, pi_hbm, mliT, mfiT, tmi_hbm, tfi_hbm, out_i)


_sc_gather = functools.partial(
    pl.kernel,
    mesh=plsc.VectorSubcoreMesh(core_axis_name="c", subcore_axis_name="s"),
    out_type=[
        jax.ShapeDtypeStruct((B, 128), jnp.float32),
        jax.ShapeDtypeStruct((B, 128), jnp.float32),
    ],
    scratch_types=[
        pltpu.VMEM((BPW + 32,), jnp.int32),
        pltpu.VMEM((PCH, 128), jnp.int32),
        pltpu.VMEM((SLOTS, DMLP, 128), jnp.float32),
        pltpu.VMEM((SLOTS, DMF, 128), jnp.float32),
        pltpu.VMEM((BPW, 128), jnp.float32),
        pltpu.VMEM((DMLP, NTAIL), jnp.float32),
        pltpu.VMEM((DMF, NTAIL), jnp.float32),
        pltpu.SemaphoreType.DMA((SLOTS,)),
        pltpu.SemaphoreType.DMA,
    ],
    compiler_params=pltpu.CompilerParams(needs_layout_passes=False,
                                         use_tc_tiling_on_sc=True),
)(_sc_gather_body)


def _mlp_body(u_ref, i_ref, w0u_ref, w0i_ref, b0_ref, w1_ref, b1_ref,
              w2_ref, b2_ref, wnm_ref, wnh_ref, bn_ref, out_ref):
    bu = u_ref[...]
    bi = i_ref[...]
    xu = bu[:, 0:DMLP]
    xi = bi[:, 0:DMLP]
    mfp = bu[:, DMLP:DMLP + DMF] * bi[:, DMLP:DMLP + DMF]
    h = jnp.dot(xu, w0u_ref[...], preferred_element_type=jnp.float32)
    h += jnp.dot(xi, w0i_ref[...], preferred_element_type=jnp.float32)
    h = jnp.maximum(h + b0_ref[...], 0.0)
    h = jnp.maximum(jnp.dot(h, w1_ref[...], preferred_element_type=jnp.float32)
                    + b1_ref[...], 0.0)
    h = jnp.maximum(jnp.dot(h, w2_ref[...], preferred_element_type=jnp.float32)
                    + b2_ref[...], 0.0)
    logit = jnp.dot(mfp, wnm_ref[...], preferred_element_type=jnp.float32)
    logit += jnp.dot(h, wnh_ref[...], preferred_element_type=jnp.float32)
    logit += bn_ref[...]
    out_ref[...] = 1.0 / (1.0 + jnp.exp(-logit))


def _mlp_call(gu, gi, w0u, w0i, b0, w1, b1, w2, b2, wnm, wnh, bn):
    BT = 2048
    grid = (B // BT,)
    row_spec = pl.BlockSpec((BT, 128), lambda i: (i, 0))
    full = lambda a, b: pl.BlockSpec((a, b), lambda i: (0, 0))
    return pl.pallas_call(
        _mlp_body,
        grid=grid,
        in_specs=[
            row_spec, row_spec,
            full(DMLP, 32), full(DMLP, 32), full(1, 32),
            full(32, 16), full(1, 16),
            full(16, 8), full(1, 8),
            full(DMF, 1), full(8, 1), full(1, 1),
        ],
        out_specs=pl.BlockSpec((BT, 1), lambda i: (i, 0)),
        out_shape=jax.ShapeDtypeStruct((B, 1), jnp.float32),
    )(gu, gi, w0u, w0i, b0, w1, b1, w2, b2, wnm, wnh, bn)


@jax.jit
def kernel(user_indices, item_indices, mf_user_table, mf_item_table,
           mlp_user_table, mlp_item_table, W0, b0, W1, b1, W2, b2, Wn, bn):
    ar = lax.iota(jnp.int32, B)
    su, pu = lax.sort((user_indices.astype(jnp.int32), ar), num_keys=1)
    si, pi = lax.sort((item_indices.astype(jnp.int32), ar), num_keys=1)
    su = su.reshape(NW, BPW)
    si = si.reshape(NW, BPW)
    gu, gi = _sc_gather(
        su, si,
        pu.reshape(B // 128, 128), pi.reshape(B // 128, 128),
        mf_user_table.T, mf_item_table.T,
        mlp_user_table.T, mlp_item_table.T,
        mf_user_table[NTB:].T, mf_item_table[NTB:].T,
        mlp_user_table[NTB:].T, mlp_item_table[NTB:].T,
    )
    return _mlp_call(gu, gi,
                     W0[:DMLP], W0[DMLP:], b0.reshape(1, 32),
                     W1, b1.reshape(1, 16), W2, b2.reshape(1, 8),
                     Wn[:DMF], Wn[DMF:], bn.reshape(1, 1))


# speedup vs baseline: 1.7237x; 1.0481x over previous
import functools

import jax
import jax.numpy as jnp
from jax import lax
from jax.experimental import pallas as pl
from jax.experimental.pallas import tpu as pltpu
from jax.experimental.pallas import tpu_sc as plsc

B = 16384
DMF = 16
DMLP = 32
NC = 2
NS = 16
NW = NC * NS
BPW = B // NW
NROWS = 1000000
NTB = (NROWS // 128) * 128
NTAIL = NROWS - NTB
RBMAX = NTB // 128 - 1
SLOTS = 4
CB = 2
CW = CB * 128
BSMAX = (NROWS - CW) // 128
PCH = BPW // 128


def _sc_gather_body(su_hbm, si_hbm, pu_hbm, pi_hbm,
                    mfuT, mfiT, mluT, mliT,
                    tfu_hbm, tfi_hbm, tmu_hbm, tmi_hbm,
                    out_u, out_i,
                    sidx_v, perm_v, fml, fmf, rowbuf, tml_v, tmf_v, sems,
                    osem):
    wid = lax.axis_index("s") * NC + lax.axis_index("c")
    rows = lax.iota(jnp.int32, 16)

    def run_phase(idx_hbm, perm_hbm, mlT, mfT, tml_hbm, tmf_hbm, out_hbm):
        pltpu.sync_copy(idx_hbm.at[wid], sidx_v.at[pl.ds(0, BPW)])
        pltpu.sync_copy(perm_hbm.at[pl.ds(wid * PCH, PCH)], perm_v)
        pltpu.sync_copy(tml_hbm, tml_v)
        pltpu.sync_copy(tmf_hbm, tmf_v)

        def fetch(k, slot):
            bs = lax.min(b0 + CB * k, BSMAX)
            off = pl.multiple_of(bs * 128, 128)
            pltpu.async_copy(mlT.at[:, pl.ds(off, CW)], fml.at[slot],
                             sems.at[slot])
            pltpu.async_copy(mfT.at[:, pl.ds(off, CW)], fmf.at[slot],
                             sems.at[slot])

        b0 = sidx_v[pl.ds(0, 16)][0] >> 7
        bl = sidx_v[pl.ds(BPW - 16, 16)][15] >> 7
        nck = (bl - b0) // CB + 1

        for k0 in range(SLOTS):
            @pl.when(k0 < nck)
            def _():
                fetch(k0, k0)

        def chunk_body(k, carry):
            p, rn = carry
            sl = lax.rem(k, SLOTS)
            pltpu.make_async_copy(mlT.at[:, pl.ds(0, CW)], fml.at[sl],
                                  sems.at[sl]).wait()
            pltpu.make_async_copy(mfT.at[:, pl.ds(0, CW)], fmf.at[sl],
                                  sems.at[sl]).wait()
            bound = (b0 + CB * (k + 1)) * 128
            bs128 = lax.min(b0 + CB * k, BSMAX) * 128
            slv = jnp.full((16,), sl, jnp.int32)

            def wcond(pr):
                p, rn = pr
                return jnp.logical_and(p < BPW, rn < bound)

            def wbody(pr):
                p, rn = pr
                rv = jnp.full((16,), rn, jnp.int32)
                lane = jnp.minimum(rv - bs128, CW - 1)
                for h in range(DMLP // 16):
                    r16 = rows + h * 16
                    rowbuf[p, pl.ds(h * 16, 16)] = plsc.load_gather(
                        fml, [slv, r16, lane])
                rowbuf[p, pl.ds(DMLP, DMF)] = plsc.load_gather(
                    fmf, [slv, rows, lane])

                @pl.when(rn >= NTB)
                def _():
                    tl = rv - NTB
                    for h in range(DMLP // 16):
                        r16 = rows + h * 16
                        rowbuf[p, pl.ds(h * 16, 16)] = plsc.load_gather(
                            tml_v, [r16, tl])
                    rowbuf[p, pl.ds(DMLP, DMF)] = plsc.load_gather(
                        tmf_v, [rows, tl])

                p2 = p + 1
                rn2 = sidx_v[pl.ds(lax.min(p2, BPW - 1), 16)][0]
                return (p2, rn2)

            p, rn = lax.while_loop(wcond, wbody, (p, rn))

            @pl.when(k + SLOTS < nck)
            def _():
                fetch(k + SLOTS, sl)
            return (p, rn)

        rn0 = sidx_v[pl.ds(0, 16)][0]
        lax.fori_loop(0, nck, chunk_body, (jnp.int32(0), rn0))

        scat = []
        for j in range(PCH):
            scat.append(pltpu.async_copy(
                rowbuf.at[pl.ds(j * 128, 128)], out_hbm.at[perm_v.at[j]],
                osem))
        for s in scat:
            s.wait()

    run_phase(su_hbm, pu_hbm, mluT, mfuT, tmu_hbm, tfu_hbm, out_u)
    run_phase(si_hbm, pi_hbm, mliT, mfiT, tmi_hbm, tfi_hbm, out_i)


_sc_gather = functools.partial(
    pl.kernel,
    mesh=plsc.VectorSubcoreMesh(core_axis_name="c", subcore_axis_name="s"),
    out_type=[
        jax.ShapeDtypeStruct((B, 128), jnp.float32),
        jax.ShapeDtypeStruct((B, 128), jnp.float32),
    ],
    scratch_types=[
        pltpu.VMEM((BPW + 32,), jnp.int32),
        pltpu.VMEM((PCH, 128), jnp.int32),
        pltpu.VMEM((SLOTS, DMLP, CW), jnp.float32),
        pltpu.VMEM((SLOTS, DMF, CW), jnp.float32),
        pltpu.VMEM((BPW, 128), jnp.float32),
        pltpu.VMEM((DMLP, NTAIL), jnp.float32),
        pltpu.VMEM((DMF, NTAIL), jnp.float32),
        pltpu.SemaphoreType.DMA((SLOTS,)),
        pltpu.SemaphoreType.DMA,
    ],
    compiler_params=pltpu.CompilerParams(needs_layout_passes=False,
                                         use_tc_tiling_on_sc=True),
)(_sc_gather_body)


def _mlp_body(u_ref, i_ref, w0u_ref, w0i_ref, b0_ref, w1_ref, b1_ref,
              w2_ref, b2_ref, wnm_ref, wnh_ref, bn_ref, out_ref):
    bu = u_ref[...]
    bi = i_ref[...]
    xu = bu[:, 0:DMLP]
    xi = bi[:, 0:DMLP]
    mfp = bu[:, DMLP:DMLP + DMF] * bi[:, DMLP:DMLP + DMF]
    h = jnp.dot(xu, w0u_ref[...], preferred_element_type=jnp.float32)
    h += jnp.dot(xi, w0i_ref[...], preferred_element_type=jnp.float32)
    h = jnp.maximum(h + b0_ref[...], 0.0)
    h = jnp.maximum(jnp.dot(h, w1_ref[...], preferred_element_type=jnp.float32)
                    + b1_ref[...], 0.0)
    h = jnp.maximum(jnp.dot(h, w2_ref[...], preferred_element_type=jnp.float32)
                    + b2_ref[...], 0.0)
    logit = jnp.dot(mfp, wnm_ref[...], preferred_element_type=jnp.float32)
    logit += jnp.dot(h, wnh_ref[...], preferred_element_type=jnp.float32)
    logit += bn_ref[...]
    out_ref[...] = 1.0 / (1.0 + jnp.exp(-logit))


def _mlp_call(gu, gi, w0u, w0i, b0, w1, b1, w2, b2, wnm, wnh, bn):
    BT = 2048
    grid = (B // BT,)
    row_spec = pl.BlockSpec((BT, 128), lambda i: (i, 0))
    full = lambda a, b: pl.BlockSpec((a, b), lambda i: (0, 0))
    return pl.pallas_call(
        _mlp_body,
        grid=grid,
        in_specs=[
            row_spec, row_spec,
            full(DMLP, 32), full(DMLP, 32), full(1, 32),
            full(32, 16), full(1, 16),
            full(16, 8), full(1, 8),
            full(DMF, 1), full(8, 1), full(1, 1),
        ],
        out_specs=pl.BlockSpec((BT, 1), lambda i: (i, 0)),
        out_shape=jax.ShapeDtypeStruct((B, 1), jnp.float32),
    )(gu, gi, w0u, w0i, b0, w1, b1, w2, b2, wnm, wnh, bn)


@jax.jit
def kernel(user_indices, item_indices, mf_user_table, mf_item_table,
           mlp_user_table, mlp_item_table, W0, b0, W1, b1, W2, b2, Wn, bn):
    ar = lax.iota(jnp.int32, B)
    su, pu = lax.sort((user_indices.astype(jnp.int32), ar), num_keys=1)
    si, pi = lax.sort((item_indices.astype(jnp.int32), ar), num_keys=1)
    su = su.reshape(NW, BPW)
    si = si.reshape(NW, BPW)
    gu, gi = _sc_gather(
        su, si,
        pu.reshape(B // 128, 128), pi.reshape(B // 128, 128),
        mf_user_table.T, mf_item_table.T,
        mlp_user_table.T, mlp_item_table.T,
        mf_user_table[NTB:].T, mf_item_table[NTB:].T,
        mlp_user_table[NTB:].T, mlp_item_table[NTB:].T,
    )
    return _mlp_call(gu, gi,
                     W0[:DMLP], W0[DMLP:], b0.reshape(1, 32),
                     W1, b1.reshape(1, 16), W2, b2.reshape(1, 8),
                     Wn[:DMF], Wn[DMF:], bn.reshape(1, 1))
